# baseline (speedup 1.0000x reference)
import jax
import jax.numpy as jnp
from jax import lax
from jax.experimental import pallas as pl
from jax.experimental.pallas import tpu as pltpu
from jax.experimental.pallas import tpu_sc as plsc

NUM_EXPERTS = 64
TOP_K = 8
BLOCK_T = 1024
_INT_MIN = -(2**31)
N_TOKENS = 32768
N_WORKERS = 32
ROWS_PER_W = N_TOKENS // N_WORKERS
PAIRS_PER_W = ROWS_PER_W // 2


def _matmul_block(x_ref, w_ref, lout_ref):
    logits_t = jax.lax.dot_general(
        w_ref[...], x_ref[...],
        (((1,), (1,)), ((), ())),
        preferred_element_type=jnp.float32,
    )
    lout_ref[...] = logits_t.T


def _tc_logits(x, W):
    n_tokens, d_model = x.shape
    return pl.pallas_call(
        _matmul_block,
        grid=(n_tokens // BLOCK_T,),
        in_specs=[
            pl.BlockSpec((BLOCK_T, d_model), lambda i: (i, 0)),
            pl.BlockSpec((NUM_EXPERTS, d_model), lambda i: (0, 0)),
        ],
        out_specs=pl.BlockSpec((BLOCK_T, NUM_EXPERTS), lambda i: (i, 0)),
        out_shape=jax.ShapeDtypeStruct((n_tokens, NUM_EXPERTS), jnp.float32),
        compiler_params=pltpu.CompilerParams(
            dimension_semantics=("arbitrary",),
        ),
    )(x, W)


def _gather16(x, idx):
    return jax.lax.gather(
        x, idx[:, None],
        jax.lax.GatherDimensionNumbers(
            offset_dims=(), collapsed_slice_dims=(0,), start_index_map=(0,)),
        slice_sizes=(1,),
        mode=jax.lax.GatherScatterMode.PROMISE_IN_BOUNDS,
    )


def _bcast_max(m):
    iota = jax.lax.iota(jnp.int32, 16)
    for s in (1, 2, 4, 8):
        m = jnp.maximum(m, _gather16(m, iota ^ s))
    return m


def _bcast_sum(m):
    iota = jax.lax.iota(jnp.int32, 16)
    for s in (1, 2, 4, 8):
        m = m + _gather16(m, iota ^ s)
    return m


def _pack_keys(vals, j):
    bits = jax.lax.bitcast_convert_type(vals, jnp.int32)
    okey = jnp.where(bits >= 0, bits, jnp.int32(_INT_MIN) - bits)
    eid = jax.lax.iota(jnp.int32, 16) + jnp.int32(16 * j)
    return (okey & jnp.int32(~63)) | (jnp.int32(63) - eid)


def _row_top8(buf, r):
    iota = jax.lax.iota(jnp.int32, 16)
    keys = [_pack_keys(buf[pl.ds(r * 64 + 16 * j, 16)], j) for j in range(4)]
    neg = jnp.full((16,), _INT_MIN, jnp.int32)
    out = jnp.zeros((16,), jnp.int32)
    mx0 = None
    for k in range(TOP_K):
        m = jnp.maximum(jnp.maximum(keys[0], keys[1]),
                        jnp.maximum(keys[2], keys[3]))
        mk = _bcast_max(m)
        if k == 0:
            mx0 = mk
        keys = [jnp.where(kk == mk, neg, kk) for kk in keys]
        out = jnp.where(iota == k, mk, out)

    idx = jnp.int32(63) - (out & jnp.int32(63))
    kq = out & jnp.int32(~63)
    vbits = jnp.where(kq >= 0, kq, jnp.int32(_INT_MIN) - kq)
    lsel = jax.lax.bitcast_convert_type(vbits, jnp.float32)
    mq = mx0 & jnp.int32(~63)
    mbits = jnp.where(mq >= 0, mq, jnp.int32(_INT_MIN) - mq)
    mxv = jax.lax.bitcast_convert_type(mbits, jnp.float32)
    e = jnp.exp(lsel - mxv)
    esel = jnp.where(iota < TOP_K, e, jnp.float32(0.0))
    return esel / _bcast_sum(esel), idx


def _sc_topk_body(logits_hbm, wout_hbm, iout_hbm, buf, wbuf, ibuf):
    wid = lax.axis_index("s") * 2 + lax.axis_index("c")
    base = wid * ROWS_PER_W
    pltpu.sync_copy(
        logits_hbm.at[pl.ds(base * NUM_EXPERTS, ROWS_PER_W * NUM_EXPERTS)],
        buf)

    def body(t, carry):
        for u in range(2):
            r = t * 2 + u
            w16, i16 = _row_top8(buf, r)
            wbuf[pl.ds(r * 16, 16)] = w16
            ibuf[pl.ds(r * 16, 16)] = i16
        return carry

    lax.fori_loop(0, PAIRS_PER_W, body, 0)
    pltpu.sync_copy(wbuf, wout_hbm.at[pl.ds(base * 16, ROWS_PER_W * 16)])
    pltpu.sync_copy(ibuf, iout_hbm.at[pl.ds(base * 16, ROWS_PER_W * 16)])


_sc_topk = pl.kernel(
    _sc_topk_body,
    out_type=[
        jax.ShapeDtypeStruct((N_TOKENS * 16,), jnp.float32),
        jax.ShapeDtypeStruct((N_TOKENS * 16,), jnp.int32),
    ],
    mesh=plsc.VectorSubcoreMesh(core_axis_name="c", subcore_axis_name="s"),
    scratch_types=[
        pltpu.VMEM((ROWS_PER_W * NUM_EXPERTS,), jnp.float32),
        pltpu.VMEM((ROWS_PER_W * 16,), jnp.float32),
        pltpu.VMEM((ROWS_PER_W * 16,), jnp.int32),
    ],
)


def kernel(x, W):
    logits = _tc_logits(x, W)
    w16, i16 = _sc_topk(logits.reshape(-1))
    w16 = w16.reshape(N_TOKENS, 16)
    i16 = i16.reshape(N_TOKENS, 16)
    return (w16[:, :TOP_K], i16[:, :TOP_K], logits)


# speedup vs baseline: 1.8377x; 1.8377x over previous
import jax
import jax.numpy as jnp
from jax.experimental import pallas as pl
from jax.experimental.pallas import tpu as pltpu

NUM_EXPERTS = 64
TOP_K = 8
BLOCK_T = 1024
_INT_MIN = -(2**31)


def _router_block(xa_ref, xb_ref, xc_ref, xd_ref, w_ref,
                  wout_ref, iout_ref, lout_ref):
    w = w_ref[...]
    kq = xa_ref.shape[1]
    logits_t = jnp.zeros((NUM_EXPERTS, xa_ref.shape[0]), jnp.float32)
    for j, xr in enumerate((xa_ref, xb_ref, xc_ref, xd_ref)):
        logits_t = logits_t + jax.lax.dot_general(
            w[:, j * kq:(j + 1) * kq], xr[...],
            (((1,), (1,)), ((), ())),
            preferred_element_type=jnp.float32,
        )
    lout_ref[...] = logits_t.T

    bt = logits_t.shape[1]
    bits = jax.lax.bitcast_convert_type(logits_t, jnp.int32)
    okey = jnp.where(bits >= 0, bits, jnp.int32(_INT_MIN) - bits)
    iota = jax.lax.broadcasted_iota(jnp.int32, (NUM_EXPERTS, bt), 0)
    key = (okey & jnp.int32(~63)) | (jnp.int32(63) - iota)

    tops = []
    for _ in range(TOP_K):
        mk = jnp.max(key, axis=0, keepdims=True)
        tops.append(mk)
        key = jnp.where(key == mk, jnp.int32(_INT_MIN), key)

    top = jnp.concatenate(tops, axis=0)
    idx = jnp.int32(63) - (top & jnp.int32(63))
    vkey = top & jnp.int32(~63)
    vbits = jnp.where(vkey >= 0, vkey, jnp.int32(_INT_MIN) - vkey)
    lsel = jax.lax.bitcast_convert_type(vbits, jnp.float32)
    e = jnp.exp(lsel - lsel[:1, :])
    wsel = e / jnp.sum(e, axis=0, keepdims=True)
    wout_ref[...] = wsel.T
    iout_ref[...] = idx.T


def kernel(x, W):
    n_tokens, d_model = x.shape
    grid = (n_tokens // BLOCK_T,)
    out_shapes = (
        jax.ShapeDtypeStruct((n_tokens, TOP_K), jnp.float32),
        jax.ShapeDtypeStruct((n_tokens, TOP_K), jnp.int32),
        jax.ShapeDtypeStruct((n_tokens, NUM_EXPERTS), jnp.float32),
    )
    return pl.pallas_call(
        _router_block,
        grid=grid,
        in_specs=[
            pl.BlockSpec((BLOCK_T, d_model // 4), lambda i: (i, 0)),
            pl.BlockSpec((BLOCK_T, d_model // 4), lambda i: (i, 1)),
            pl.BlockSpec((BLOCK_T, d_model // 4), lambda i: (i, 2)),
            pl.BlockSpec((BLOCK_T, d_model // 4), lambda i: (i, 3)),
            pl.BlockSpec((NUM_EXPERTS, d_model), lambda i: (0, 0)),
        ],
        out_specs=(
            pl.BlockSpec((BLOCK_T, TOP_K), lambda i: (i, 0)),
            pl.BlockSpec((BLOCK_T, TOP_K), lambda i: (i, 0)),
            pl.BlockSpec((BLOCK_T, NUM_EXPERTS), lambda i: (i, 0)),
        ),
        out_shape=out_shapes,
        compiler_params=pltpu.CompilerParams(
            dimension_semantics=("parallel",),
        ),
    )(x, x, x, x, W)
